# baseline (speedup 1.0000x reference)
import functools

import jax
import jax.numpy as jnp
from jax.experimental import pallas as pl
from jax.experimental.pallas import tpu as pltpu

_V = 8192
_N = 8192


def _body(idx_ref, tgt_ref, table_blk, out_blk, loss_ref, acc_ref):
    i = pl.program_id(0)
    blk = table_blk[...]
    out_blk[...] = blk
    m = jnp.max(blk)
    s = jnp.sum(jnp.exp(blk - m))
    lse = m + jnp.log(s)
    t = tgt_ref[i]
    col = jax.lax.broadcasted_iota(jnp.int32, (1, 1, _V), 2)
    xt = jnp.sum(jnp.where(col == t, blk, 0.0))

    @pl.when(i == 0)
    def _init():
        acc_ref[0] = 0.0

    acc_ref[0] += lse - xt

    @pl.when(i == _N - 1)
    def _fin():
        loss_ref[0, 0] = acc_ref[0] / _N


def kernel(inputs, targets, table):
    idx = inputs.reshape(-1)
    tgt = targets.reshape(-1)

    grid_spec = pltpu.PrefetchScalarGridSpec(
        num_scalar_prefetch=2,
        grid=(_N,),
        in_specs=[
            pl.BlockSpec((1, 1, _V), lambda i, idx_ref, tgt_ref: (idx_ref[i], 0, 0)),
        ],
        out_specs=[
            pl.BlockSpec((1, 1, _V), lambda i, idx_ref, tgt_ref: (i, 0, 0)),
            pl.BlockSpec(memory_space=pltpu.SMEM),
        ],
        scratch_shapes=[pltpu.SMEM((1,), jnp.float32)],
    )

    logits, loss = pl.pallas_call(
        _body,
        grid_spec=grid_spec,
        out_shape=[
            jax.ShapeDtypeStruct((_N, 1, _V), jnp.float32),
            jax.ShapeDtypeStruct((1, 1), jnp.float32),
        ],
    )(idx, tgt, table.reshape(_V, 1, _V))
    return logits.reshape(_N, _V), loss[0, 0]


# speedup vs baseline: 11.7495x; 11.7495x over previous
import functools

import jax
import jax.numpy as jnp
from jax import lax
from jax.experimental import pallas as pl
from jax.experimental.pallas import tpu as pltpu
from jax.experimental.pallas import tpu_sc as plsc

_V = 8192
_N = 8192
_NC = 2
_NS = 16
_NW = _NC * _NS
_RPW = _N // _NW
_C = 4
_NCH = _RPW // _C

_mesh = plsc.VectorSubcoreMesh(core_axis_name="c", subcore_axis_name="s")


def _worker_id():
    return lax.axis_index("s") * _NC + lax.axis_index("c")


@functools.partial(
    pl.kernel,
    mesh=_mesh,
    out_type=jax.ShapeDtypeStruct((_N, _V), jnp.float32),
    scratch_types=[
        pltpu.VMEM((_NCH, _C), jnp.int32),
        pltpu.VMEM((_C, _V), jnp.float32),
        pltpu.VMEM((_C, _V), jnp.float32),
        pltpu.SemaphoreType.DMA,
        pltpu.SemaphoreType.DMA,
        pltpu.SemaphoreType.DMA,
        pltpu.SemaphoreType.DMA,
    ],
)
def _gather_rows(table_hbm, idx_hbm, out_hbm, idx_v, buf0, buf1, g0, g1, s0, s1):
    wid = _worker_id()
    base = wid * _RPW
    pltpu.sync_copy(idx_hbm.at[wid], idx_v)

    bufs = (buf0, buf1)
    gsems = (g0, g1)
    ssems = (s0, s1)

    def g_src(g):
        return table_hbm.at[idx_v.at[g]]

    def s_dst(g):
        return out_hbm.at[pl.ds(base + g * _C, _C)]

    def start_gather(g, b):
        pltpu.make_async_copy(g_src(g), bufs[b], gsems[b]).start()

    def wait_gather(g, b):
        pltpu.make_async_copy(g_src(g), bufs[b], gsems[b]).wait()

    def start_scatter(g, b):
        pltpu.make_async_copy(bufs[b], s_dst(g), ssems[b]).start()

    def wait_scatter(g, b):
        pltpu.make_async_copy(bufs[b], s_dst(g), ssems[b]).wait()

    start_gather(0, 0)
    wait_gather(0, 0)
    start_scatter(0, 0)
    start_gather(1, 1)

    def body(j, carry):
        ga = 2 * j + 1
        wait_gather(ga, 1)
        start_scatter(ga, 1)
        wait_scatter(ga - 1, 0)
        start_gather(ga + 1, 0)
        gb = 2 * j + 2
        wait_gather(gb, 0)
        start_scatter(gb, 0)
        wait_scatter(gb - 1, 1)
        start_gather(gb + 1, 1)
        return carry

    lax.fori_loop(0, (_NCH - 2) // 2, body, 0)

    gl = _NCH - 1
    wait_gather(gl, 1)
    start_scatter(gl, 1)
    wait_scatter(gl - 1, 0)
    wait_scatter(gl, 1)


_BR = 256


def _lse_body(tbl_ref, lse_ref):
    blk = tbl_ref[...]
    m = jnp.max(blk, axis=1)
    s = jnp.sum(jnp.exp(blk - m[:, None]), axis=1)
    lse_ref[0, 0, :] = m + jnp.log(s)


def _table_lse(table):
    lse3 = pl.pallas_call(
        _lse_body,
        grid=(_V // _BR,),
        in_specs=[pl.BlockSpec((_BR, _V), lambda i: (i, 0))],
        out_specs=pl.BlockSpec((1, 1, _BR), lambda i: (i, 0, 0)),
        out_shape=jax.ShapeDtypeStruct((_V // _BR, 1, _BR), jnp.float32),
    )(table)
    return lse3.reshape(_V)


@functools.partial(
    pl.kernel,
    mesh=_mesh,
    out_type=jax.ShapeDtypeStruct((_NW, 16), jnp.float32),
    scratch_types=[
        pltpu.VMEM((_RPW,), jnp.int32),
        pltpu.VMEM((_RPW,), jnp.int32),
        pltpu.VMEM((_RPW,), jnp.int32),
        pltpu.VMEM((_RPW,), jnp.float32),
        pltpu.VMEM((_RPW,), jnp.float32),
        pltpu.VMEM((16,), jnp.float32),
        pltpu.SemaphoreType.DMA,
    ],
)
def _nll_partials(
    tablef_hbm, idx_hbm, tgt_hbm, lse_hbm, out_hbm,
    idx_v, tgt_v, fl_v, xt_v, lse_g, part_v, sem,
):
    wid = _worker_id()
    base = wid * _RPW
    pltpu.sync_copy(idx_hbm.at[pl.ds(base, _RPW)], idx_v)
    pltpu.sync_copy(tgt_hbm.at[pl.ds(base, _RPW)], tgt_v)

    for k in range(_RPW // 16):
        sl = pl.ds(k * 16, 16)
        fl_v[sl] = idx_v[sl] * _V + tgt_v[sl]

    for h in range(_RPW // 128):
        sl = pl.ds(h * 128, 128)
        pltpu.make_async_copy(
            tablef_hbm.at[fl_v.at[sl]], xt_v.at[sl], sem
        ).start()
        pltpu.make_async_copy(
            tablef_hbm.at[fl_v.at[sl]], xt_v.at[sl], sem
        ).wait()
        pltpu.make_async_copy(
            lse_hbm.at[idx_v.at[sl]], lse_g.at[sl], sem
        ).start()
        pltpu.make_async_copy(
            lse_hbm.at[idx_v.at[sl]], lse_g.at[sl], sem
        ).wait()

    acc = jnp.zeros((16,), jnp.float32)
    for k in range(_RPW // 16):
        sl = pl.ds(k * 16, 16)
        acc = acc + (lse_g[sl] - xt_v[sl])

    part_v[...] = acc
    pltpu.sync_copy(part_v, out_hbm.at[wid])


def kernel(inputs, targets, table):
    idx = inputs.reshape(-1)
    tgt = targets.reshape(-1)

    logits = _gather_rows(table, idx.reshape(_NW, _NCH, _C))
    lse = _table_lse(table)
    partials = _nll_partials(table.reshape(_V * _V), idx, tgt, lse)
    loss = jnp.sum(partials) / _N
    return logits, loss


# speedup vs baseline: 19.2612x; 1.6393x over previous
import functools

import jax
import jax.numpy as jnp
from jax import lax
from jax.experimental import pallas as pl
from jax.experimental.pallas import tpu as pltpu
from jax.experimental.pallas import tpu_sc as plsc

_V = 8192
_N = 8192
_NC = 2
_NS = 16
_NW = _NC * _NS
_RPW = _N // _NW
_C = 4
_NCH = _RPW // _C

_mesh = plsc.VectorSubcoreMesh(core_axis_name="c", subcore_axis_name="s")


def _worker_id():
    return lax.axis_index("s") * _NC + lax.axis_index("c")


@functools.partial(
    pl.kernel,
    mesh=_mesh,
    out_type=[
        jax.ShapeDtypeStruct((_N, _V), jnp.float32),
        jax.ShapeDtypeStruct((_NW, 16), jnp.float32),
    ],
    scratch_types=[
        pltpu.VMEM((_NCH, _C), jnp.int32),
        pltpu.VMEM((_RPW + 16,), jnp.int32),
        pltpu.VMEM((_C, _V), jnp.float32),
        pltpu.VMEM((_C, _V), jnp.float32),
        pltpu.VMEM((16,), jnp.float32),
        pltpu.SemaphoreType.DMA,
        pltpu.SemaphoreType.DMA,
        pltpu.SemaphoreType.DMA,
        pltpu.SemaphoreType.DMA,
    ],
)
def _gather_rows(
    table_hbm, idx_hbm, tgt_hbm, out_hbm, xt_hbm,
    idx_v, tgt_v, buf0, buf1, part_v, g0, g1, s0, s1,
):
    wid = _worker_id()
    base = wid * _RPW
    pltpu.sync_copy(idx_hbm.at[wid], idx_v)
    pltpu.sync_copy(tgt_hbm.at[pl.ds(base, _RPW)], tgt_v.at[pl.ds(0, _RPW)])

    bufs = (buf0, buf1)
    gsems = (g0, g1)
    ssems = (s0, s1)

    def g_src(g):
        return table_hbm.at[idx_v.at[g]]

    def s_dst(g):
        return out_hbm.at[pl.ds(base + g * _C, _C)]

    def start_gather(g, b):
        pltpu.make_async_copy(g_src(g), bufs[b], gsems[b]).start()

    def wait_gather(g, b):
        pltpu.make_async_copy(g_src(g), bufs[b], gsems[b]).wait()

    def start_scatter(g, b):
        pltpu.make_async_copy(bufs[b], s_dst(g), ssems[b]).start()

    def wait_scatter(g, b):
        pltpu.make_async_copy(bufs[b], s_dst(g), ssems[b]).wait()

    lanes = lax.iota(jnp.int32, 16)

    def extract_xt(g, b, acc, g_parity):
        for r in range(_C):
            s = g * _C + r
            off = pl.multiple_of((s >> 3) * 8, 8)
            v = tgt_v[pl.ds(off, 16)]
            t = v[(_C * g_parity + r) & 7]
            t_hi = pl.multiple_of(t & ~15, 16)
            vec = bufs[b][r, pl.ds(t_hi, 16)]
            acc = acc + jnp.where(lanes == (t & 15), vec, 0.0)
        return acc

    start_gather(0, 0)
    wait_gather(0, 0)
    start_scatter(0, 0)
    start_gather(1, 1)
    acc0 = extract_xt(0, 0, jnp.zeros((16,), jnp.float32), 0)

    def body(j, acc):
        ga = 2 * j + 1
        wait_gather(ga, 1)
        start_scatter(ga, 1)
        wait_scatter(ga - 1, 0)
        start_gather(ga + 1, 0)
        acc = extract_xt(ga, 1, acc, 1)
        gb = 2 * j + 2
        wait_gather(gb, 0)
        start_scatter(gb, 0)
        wait_scatter(gb - 1, 1)
        start_gather(gb + 1, 1)
        acc = extract_xt(gb, 0, acc, 0)
        return acc

    acc = lax.fori_loop(0, (_NCH - 2) // 2, body, acc0)

    gl = _NCH - 1
    wait_gather(gl, 1)
    start_scatter(gl, 1)
    wait_scatter(gl - 1, 0)
    acc = extract_xt(gl, 1, acc, 1)
    wait_scatter(gl, 1)

    part_v[...] = acc
    pltpu.sync_copy(part_v, xt_hbm.at[wid])


_BR = 256


def _lse_body(tbl_ref, lse_ref):
    blk = tbl_ref[...]
    m = jnp.max(blk, axis=1)
    s = jnp.sum(jnp.exp(blk - m[:, None]), axis=1)
    lse_ref[0, 0, :] = m + jnp.log(s)


def _table_lse(table):
    lse3 = pl.pallas_call(
        _lse_body,
        grid=(_V // _BR,),
        in_specs=[pl.BlockSpec((_BR, _V), lambda i: (i, 0))],
        out_specs=pl.BlockSpec((1, 1, _BR), lambda i: (i, 0, 0)),
        out_shape=jax.ShapeDtypeStruct((_V // _BR, 1, _BR), jnp.float32),
    )(table)
    return lse3.reshape(_V)


@functools.partial(
    pl.kernel,
    mesh=_mesh,
    out_type=jax.ShapeDtypeStruct((_NW, 16), jnp.float32),
    scratch_types=[
        pltpu.VMEM((_RPW,), jnp.int32),
        pltpu.VMEM((_RPW,), jnp.float32),
        pltpu.VMEM((16,), jnp.float32),
        pltpu.SemaphoreType.DMA,
    ],
)
def _lse_partials(idx_hbm, lse_hbm, out_hbm, idx_v, lse_g, part_v, sem):
    wid = _worker_id()
    base = wid * _RPW
    pltpu.sync_copy(idx_hbm.at[pl.ds(base, _RPW)], idx_v)

    for h in range(_RPW // 128):
        sl = pl.ds(h * 128, 128)
        pltpu.make_async_copy(
            lse_hbm.at[idx_v.at[sl]], lse_g.at[sl], sem
        ).start()
        pltpu.make_async_copy(
            lse_hbm.at[idx_v.at[sl]], lse_g.at[sl], sem
        ).wait()

    acc = jnp.zeros((16,), jnp.float32)
    for k in range(_RPW // 16):
        acc = acc + lse_g[pl.ds(k * 16, 16)]

    part_v[...] = acc
    pltpu.sync_copy(part_v, out_hbm.at[wid])


def kernel(inputs, targets, table):
    idx = inputs.reshape(_NW, _NCH, _C)

    logits, xt_parts = _gather_rows(table, idx, targets.reshape(-1))
    lse = _table_lse(table)
    lse_parts = _lse_partials(inputs.reshape(-1), lse)
    loss = (jnp.sum(lse_parts) - jnp.sum(xt_parts)) / _N
    return logits, loss
